# Initial kernel scaffold; baseline (speedup 1.0000x reference)
import functools

import jax
import jax.numpy as jnp
from jax import lax
from jax.experimental import pallas as pl
from jax.experimental.pallas import tpu as pltpu
from jax.experimental.pallas import tpu_sc as plsc

N = 50000
NPAD = 51200
E = 800000
EPAD = 802816
NB = 392
NBH = 196
ROWS_PER_TILE = NPAD // 16
BLK = 512
GRID = NPAD // BLK

_mesh = plsc.VectorSubcoreMesh(core_axis_name="c", subcore_axis_name="s")


@functools.partial(
    pl.kernel,
    out_type=jax.ShapeDtypeStruct((2, NPAD, 8), jnp.float32),
    mesh=_mesh,
    scratch_types=[
        pltpu.VMEM((NBH, 128), jnp.int32),
        pltpu.VMEM((128, 8), jnp.float32),
        pltpu.VMEM_SHARED((NPAD, 8), jnp.float32),
    ],
)
def _deg_kernel(dst_hbm, ones_hbm, zeros_hbm, out_hbm, idx_v, ones_v, acc):
    c = lax.axis_index("c")
    s = lax.axis_index("s")
    w = s * 2 + c
    pltpu.sync_copy(zeros_hbm, acc.at[pl.ds(s * ROWS_PER_TILE, ROWS_PER_TILE)])
    pltpu.sync_copy(ones_hbm, ones_v)
    pltpu.sync_copy(dst_hbm.at[w], idx_v)
    plsc.subcore_barrier()

    def body(j, carry):
        pltpu.sync_copy(ones_v, acc.at[idx_v.at[j]], add=True)
        return carry

    lax.fori_loop(0, NBH, body, 0)
    plsc.subcore_barrier()
    sl = pl.ds(s * ROWS_PER_TILE, ROWS_PER_TILE)
    pltpu.sync_copy(acc.at[sl], out_hbm.at[c, sl])


@functools.partial(
    pl.kernel,
    out_type=jax.ShapeDtypeStruct((4, NPAD, 32), jnp.float32),
    mesh=_mesh,
    scratch_types=[
        pltpu.VMEM((NBH, 128), jnp.int32),
        pltpu.VMEM((NBH, 128), jnp.int32),
        pltpu.VMEM((128, 32), jnp.float32),
        pltpu.VMEM_SHARED((NPAD, 32), jnp.float32),
        pltpu.SemaphoreType.DMA,
    ],
)
def _seg_kernel(gt_hbm, src_hbm, dst_hbm, zeros_hbm, out_hbm,
                src_v, dst_v, rows_v, acc, gsem):
    c = lax.axis_index("c")
    s = lax.axis_index("s")
    sl = pl.ds(s * ROWS_PER_TILE, ROWS_PER_TILE)
    for k in range(2):
        chunk = c * 2 + k
        pltpu.sync_copy(zeros_hbm, acc.at[sl])
        plsc.subcore_barrier()
        for h in range(2):
            pltpu.sync_copy(src_hbm.at[chunk, s, pl.ds(h * NBH, NBH)], src_v)
            pltpu.sync_copy(dst_hbm.at[s, pl.ds(h * NBH, NBH)], dst_v)

            def body(j, carry):
                pltpu.async_copy(gt_hbm.at[src_v.at[j]], rows_v, gsem).wait()
                pltpu.sync_copy(rows_v, acc.at[dst_v.at[j]], add=True)
                return carry

            lax.fori_loop(0, NBH, body, 0)
        plsc.subcore_barrier()
        pltpu.sync_copy(acc.at[sl], out_hbm.at[chunk, sl])


def _mm_scale_body(z_ref, w_ref, deg_ref, dinv_ref, g_ref):
    d = deg_ref[...]
    dinv = lax.rsqrt(d[0, :, :1] + d[1, :, :1] + 1.0)
    dinv_ref[...] = jnp.broadcast_to(dinv, (BLK, 8))
    h = jnp.dot(z_ref[...], w_ref[...], preferred_element_type=jnp.float32)
    g = h * dinv
    for cch in range(4):
        g_ref[cch, :, :] = g[:, cch * 32:(cch + 1) * 32]


def _layer2_body(s_ref, g_ref, dinv_ref, b_ref, w_ref, g2_ref):
    dinv = dinv_ref[:, :1]
    sc = jnp.concatenate([s_ref[i] for i in range(4)], axis=1)
    gc = jnp.concatenate([g_ref[i] for i in range(4)], axis=1)
    x = jnp.maximum((sc + gc) * dinv + b_ref[...], 0.0)
    h2 = jnp.dot(x, w_ref[...], preferred_element_type=jnp.float32)
    g2 = h2 * dinv
    for cch in range(4):
        g2_ref[cch, :, :] = g2[:, cch * 32:(cch + 1) * 32]


def _final_body(s_ref, g_ref, dinv_ref, b_ref, o_ref):
    dinv = dinv_ref[:, :1]
    sc = jnp.concatenate([s_ref[i] for i in range(4)], axis=1)
    gc = jnp.concatenate([g_ref[i] for i in range(4)], axis=1)
    o_ref[...] = (sc + gc) * dinv + b_ref[...]


def _chunk_spec():
    return pl.BlockSpec((4, BLK, 32), lambda i: (0, i, 0))


def kernel(z, edge_index, W1, b1, W2, b2):
    f32 = jnp.float32
    z_p = jnp.zeros((NPAD, z.shape[1]), f32).at[:N].set(z)
    ei = jnp.pad(edge_index, ((0, 0), (0, EPAD - E)), constant_values=N)
    src = ei[0]
    dst = ei[1]
    dst16 = dst.reshape(16, NB, 128)
    dst32 = dst.reshape(32, NBH, 128)
    src4 = (src.reshape(1, 16, NB, 128)
            + (jnp.arange(4, dtype=jnp.int32) * NPAD).reshape(4, 1, 1, 1))

    ones8 = jnp.ones((128, 8), f32)
    zeros8 = jnp.zeros((ROWS_PER_TILE, 8), f32)
    zeros32 = jnp.zeros((ROWS_PER_TILE, 32), f32)

    deg8 = _deg_kernel(dst32, ones8, zeros8)

    dinv8, g1 = pl.pallas_call(
        _mm_scale_body,
        grid=(GRID,),
        in_specs=[
            pl.BlockSpec((BLK, 64), lambda i: (i, 0)),
            pl.BlockSpec((64, 128), lambda i: (0, 0)),
            pl.BlockSpec((2, BLK, 8), lambda i: (0, i, 0)),
        ],
        out_specs=[
            pl.BlockSpec((BLK, 8), lambda i: (i, 0)),
            _chunk_spec(),
        ],
        out_shape=[
            jax.ShapeDtypeStruct((NPAD, 8), f32),
            jax.ShapeDtypeStruct((4, NPAD, 32), f32),
        ],
    )(z_p, W1, deg8)

    s1 = _seg_kernel(g1.reshape(4 * NPAD, 32), src4, dst16, zeros32)

    g2 = pl.pallas_call(
        _layer2_body,
        grid=(GRID,),
        in_specs=[
            _chunk_spec(),
            _chunk_spec(),
            pl.BlockSpec((BLK, 8), lambda i: (i, 0)),
            pl.BlockSpec((1, 128), lambda i: (0, 0)),
            pl.BlockSpec((128, 128), lambda i: (0, 0)),
        ],
        out_specs=_chunk_spec(),
        out_shape=jax.ShapeDtypeStruct((4, NPAD, 32), f32),
    )(s1, g1, dinv8, b1.reshape(1, 128), W2)

    s2 = _seg_kernel(g2.reshape(4 * NPAD, 32), src4, dst16, zeros32)

    x_hat = pl.pallas_call(
        _final_body,
        grid=(GRID,),
        in_specs=[
            _chunk_spec(),
            _chunk_spec(),
            pl.BlockSpec((BLK, 8), lambda i: (i, 0)),
            pl.BlockSpec((1, 128), lambda i: (0, 0)),
        ],
        out_specs=pl.BlockSpec((BLK, 128), lambda i: (i, 0)),
        out_shape=jax.ShapeDtypeStruct((NPAD, 128), f32),
    )(s2, g2, dinv8, b2.reshape(1, 128))

    return x_hat[:N]


# speedup vs baseline: 3.3057x; 3.3057x over previous
import functools

import jax
import jax.numpy as jnp
from jax import lax
from jax.experimental import pallas as pl
from jax.experimental.pallas import tpu as pltpu
from jax.experimental.pallas import tpu_sc as plsc

N = 50000
NPAD = 51200
E = 800000
EPAD = 802816
NB = 392
ROWS_PER_TILE = NPAD // 16
BLK = 512
GRID = NPAD // BLK
NCH = 16
CW = 128 // NCH

_mesh = plsc.VectorSubcoreMesh(core_axis_name="c", subcore_axis_name="s")


@functools.partial(
    pl.kernel,
    out_type=jax.ShapeDtypeStruct((2, NPAD, 4), jnp.float32),
    mesh=_mesh,
    scratch_types=[
        pltpu.VMEM((NB, 128), jnp.int32),
        pltpu.VMEM((128, 4), jnp.float32),
        pltpu.VMEM_SHARED((NPAD, 4), jnp.float32),
    ],
    compiler_params=pltpu.CompilerParams(use_tc_tiling_on_sc=False),
)
def _deg_kernel(dst_hbm, ones_hbm, zeros_hbm, out_hbm, idx_v, ones_v, acc):
    c = lax.axis_index("c")
    s = lax.axis_index("s")
    pltpu.sync_copy(zeros_hbm, acc.at[pl.ds(s * ROWS_PER_TILE, ROWS_PER_TILE)])
    pltpu.sync_copy(ones_hbm, ones_v)
    pltpu.sync_copy(dst_hbm.at[s], idx_v)
    plsc.subcore_barrier()

    def body(j, carry):
        pltpu.sync_copy(ones_v, acc.at[idx_v.at[j]], add=True)
        return carry

    lax.fori_loop(0, NB, body, 0)
    plsc.subcore_barrier()
    sl = pl.ds(s * ROWS_PER_TILE, ROWS_PER_TILE)
    pltpu.sync_copy(acc.at[sl], out_hbm.at[c, sl])


@functools.partial(
    pl.kernel,
    out_type=jax.ShapeDtypeStruct((NCH, NPAD, CW), jnp.float32),
    mesh=_mesh,
    scratch_types=[
        pltpu.VMEM((NB, 128), jnp.int32),
        pltpu.VMEM((NB, 128), jnp.int32),
        pltpu.VMEM((128, CW), jnp.float32),
        pltpu.VMEM_SHARED((NPAD, CW), jnp.float32),
        pltpu.SemaphoreType.DMA,
    ],
    compiler_params=pltpu.CompilerParams(use_tc_tiling_on_sc=False),
)
def _seg_kernel(gt_hbm, src_hbm, dst_hbm, zeros_hbm, out_hbm,
                src_v, dst_v, rows_v, acc, gsem):
    c = lax.axis_index("c")
    s = lax.axis_index("s")
    sl = pl.ds(s * ROWS_PER_TILE, ROWS_PER_TILE)
    pltpu.sync_copy(src_hbm.at[s], src_v)
    pltpu.sync_copy(dst_hbm.at[s], dst_v)
    for k in range(NCH // 2):
        chunk = c * (NCH // 2) + k
        pltpu.sync_copy(zeros_hbm, acc.at[sl])
        plsc.subcore_barrier()

        def body(j, carry):
            pltpu.async_copy(gt_hbm.at[chunk].at[src_v.at[j]], rows_v,
                             gsem).wait()
            pltpu.sync_copy(rows_v, acc.at[dst_v.at[j]], add=True)
            return carry

        lax.fori_loop(0, NB, body, 0)
        plsc.subcore_barrier()
        pltpu.sync_copy(acc.at[sl], out_hbm.at[chunk, sl])


def _mm_scale_body(z_ref, w_ref, deg_ref, dinv_ref, g_ref):
    d = deg_ref[...]
    dinv = lax.rsqrt(d[0, :, :1] + 1.0)
    dinv_ref[...] = jnp.broadcast_to(dinv, (BLK, 8))
    h = jnp.dot(z_ref[...], w_ref[...], preferred_element_type=jnp.float32)
    g = h * dinv
    for cch in range(NCH):
        g_ref[cch, :, :] = g[:, cch * CW:(cch + 1) * CW]


def _layer2_body(s_ref, g_ref, dinv_ref, b_ref, w_ref, g2_ref):
    dinv = dinv_ref[:, :1]
    sc = jnp.concatenate([s_ref[i] for i in range(NCH)], axis=1)
    gc = jnp.concatenate([g_ref[i] for i in range(NCH)], axis=1)
    x = jnp.maximum((sc + gc) * dinv + b_ref[...], 0.0)
    h2 = jnp.dot(x, w_ref[...], preferred_element_type=jnp.float32)
    g2 = h2 * dinv
    for cch in range(NCH):
        g2_ref[cch, :, :] = g2[:, cch * CW:(cch + 1) * CW]


def _final_body(s_ref, g_ref, dinv_ref, b_ref, o_ref):
    dinv = dinv_ref[:, :1]
    sc = jnp.concatenate([s_ref[i] for i in range(NCH)], axis=1)
    gc = jnp.concatenate([g_ref[i] for i in range(NCH)], axis=1)
    o_ref[...] = (sc + gc) * dinv + b_ref[...]


def _chunk_spec():
    return pl.BlockSpec((NCH, BLK, CW), lambda i: (0, i, 0))


def kernel(z, edge_index, W1, b1, W2, b2):
    f32 = jnp.float32
    z_p = jnp.zeros((NPAD, z.shape[1]), f32).at[:N].set(z)
    ei = jnp.pad(edge_index, ((0, 0), (0, EPAD - E)), constant_values=N)
    src = ei[0]
    dst = ei[1]
    dst16 = dst.reshape(16, NB, 128)
    src16 = src.reshape(16, NB, 128)

    ones4 = jnp.ones((128, 4), f32)
    zeros4 = jnp.zeros((ROWS_PER_TILE, 4), f32)
    zeros_cw = jnp.zeros((ROWS_PER_TILE, CW), f32)

    deg4 = _deg_kernel(dst16, ones4, zeros4)

    dinv8, g1 = pl.pallas_call(
        _mm_scale_body,
        grid=(GRID,),
        in_specs=[
            pl.BlockSpec((BLK, 64), lambda i: (i, 0)),
            pl.BlockSpec((64, 128), lambda i: (0, 0)),
            pl.BlockSpec((2, BLK, 4), lambda i: (0, i, 0)),
        ],
        out_specs=[
            pl.BlockSpec((BLK, 8), lambda i: (i, 0)),
            _chunk_spec(),
        ],
        out_shape=[
            jax.ShapeDtypeStruct((NPAD, 8), f32),
            jax.ShapeDtypeStruct((NCH, NPAD, CW), f32),
        ],
    )(z_p, W1, deg4)

    s1 = _seg_kernel(g1, src16, dst16, zeros_cw)

    g2 = pl.pallas_call(
        _layer2_body,
        grid=(GRID,),
        in_specs=[
            _chunk_spec(),
            _chunk_spec(),
            pl.BlockSpec((BLK, 8), lambda i: (i, 0)),
            pl.BlockSpec((1, 128), lambda i: (0, 0)),
            pl.BlockSpec((128, 128), lambda i: (0, 0)),
        ],
        out_specs=_chunk_spec(),
        out_shape=jax.ShapeDtypeStruct((NCH, NPAD, CW), f32),
    )(s1, g1, dinv8, b1.reshape(1, 128), W2)

    s2 = _seg_kernel(g2, src16, dst16, zeros_cw)

    x_hat = pl.pallas_call(
        _final_body,
        grid=(GRID,),
        in_specs=[
            _chunk_spec(),
            _chunk_spec(),
            pl.BlockSpec((BLK, 8), lambda i: (i, 0)),
            pl.BlockSpec((1, 128), lambda i: (0, 0)),
        ],
        out_specs=pl.BlockSpec((BLK, 128), lambda i: (i, 0)),
        out_shape=jax.ShapeDtypeStruct((NPAD, 128), f32),
    )(s2, g2, dinv8, b2.reshape(1, 128))

    return x_hat[:N]


# speedup vs baseline: 5.3506x; 1.6186x over previous
import functools

import jax
import jax.numpy as jnp
from jax import lax
from jax.experimental import pallas as pl
from jax.experimental.pallas import tpu as pltpu
from jax.experimental.pallas import tpu_sc as plsc

N = 50000
NPAD = 50176
E = 800000
EPAD = 802816
NB = 392
ROWS_PER_TILE = NPAD // 16
BLK = 512
GRID = NPAD // BLK
NCH = 16
CW = 128 // NCH
RING = 4
NGRP = NB // RING

_mesh = plsc.VectorSubcoreMesh(core_axis_name="c", subcore_axis_name="s")


@functools.partial(
    pl.kernel,
    out_type=jax.ShapeDtypeStruct((2, NPAD, 1), jnp.float32),
    mesh=_mesh,
    scratch_types=[
        pltpu.VMEM((NB, 128), jnp.int32),
        pltpu.VMEM((128, 1), jnp.float32),
        pltpu.VMEM_SHARED((NPAD, 1), jnp.float32),
    ],
    compiler_params=pltpu.CompilerParams(use_tc_tiling_on_sc=False),
)
def _deg_kernel(dst_hbm, ones_hbm, zeros_hbm, out_hbm, idx_v, ones_v, acc):
    c = lax.axis_index("c")
    s = lax.axis_index("s")
    pltpu.sync_copy(zeros_hbm, acc.at[pl.ds(s * ROWS_PER_TILE, ROWS_PER_TILE)])
    pltpu.sync_copy(ones_hbm, ones_v)
    pltpu.sync_copy(dst_hbm.at[s], idx_v)
    plsc.subcore_barrier()

    def body(j, carry):
        pltpu.sync_copy(ones_v, acc.at[idx_v.at[j]], add=True)
        return carry

    lax.fori_loop(0, NB, body, 0)
    plsc.subcore_barrier()
    sl = pl.ds(s * ROWS_PER_TILE, ROWS_PER_TILE)
    pltpu.sync_copy(acc.at[sl], out_hbm.at[c, sl])


@functools.partial(
    pl.kernel,
    out_type=jax.ShapeDtypeStruct((NCH, NPAD, CW), jnp.float32),
    mesh=_mesh,
    scratch_types=[
        pltpu.VMEM((NB, 128), jnp.int32),
        pltpu.VMEM((NB, 128), jnp.int32),
    ] + [pltpu.VMEM((128, CW), jnp.float32) for _ in range(RING)] + [
        pltpu.VMEM_SHARED((NPAD, CW), jnp.float32),
        pltpu.SemaphoreType.DMA,
    ],
    compiler_params=pltpu.CompilerParams(use_tc_tiling_on_sc=False),
)
def _seg_kernel(gt_hbm, src_hbm, dst_hbm, zeros_hbm, out_hbm,
                src_v, dst_v, *rest):
    rows = rest[:RING]
    acc, gsem = rest[RING], rest[RING + 1]
    c = lax.axis_index("c")
    s = lax.axis_index("s")
    sl = pl.ds(s * ROWS_PER_TILE, ROWS_PER_TILE)
    pltpu.sync_copy(src_hbm.at[s], src_v)
    pltpu.sync_copy(dst_hbm.at[s], dst_v)
    for k in range(NCH // 2):
        chunk = c * (NCH // 2) + k
        pltpu.sync_copy(zeros_hbm, acc.at[sl])
        plsc.subcore_barrier()

        def grp(g, carry):
            ds = [pltpu.async_copy(gt_hbm.at[chunk].at[src_v.at[g * RING + r]],
                                   rows[r], gsem) for r in range(RING)]
            for r in range(RING):
                ds[r].wait()
                pltpu.sync_copy(rows[r], acc.at[dst_v.at[g * RING + r]],
                                add=True)
            return carry

        lax.fori_loop(0, NGRP, grp, 0)
        plsc.subcore_barrier()
        pltpu.sync_copy(acc.at[sl], out_hbm.at[chunk, sl])


def _mm_scale_body(z_ref, w_ref, deg_ref, dinv_ref, g_ref):
    d = deg_ref[...]
    dinv = lax.rsqrt(d[0, :, :1] + 1.0)
    dinv_ref[...] = jnp.broadcast_to(dinv, (BLK, 8))
    h = jnp.dot(z_ref[...], w_ref[...], preferred_element_type=jnp.float32)
    g = h * dinv
    for cch in range(NCH):
        g_ref[cch, :, :] = g[:, cch * CW:(cch + 1) * CW]


def _layer2_body(s_ref, g_ref, dinv_ref, b_ref, w_ref, g2_ref):
    dinv = dinv_ref[:, :1]
    sc = jnp.concatenate([s_ref[i] for i in range(NCH)], axis=1)
    gc = jnp.concatenate([g_ref[i] for i in range(NCH)], axis=1)
    x = jnp.maximum((sc + gc) * dinv + b_ref[...], 0.0)
    h2 = jnp.dot(x, w_ref[...], preferred_element_type=jnp.float32)
    g2 = h2 * dinv
    for cch in range(NCH):
        g2_ref[cch, :, :] = g2[:, cch * CW:(cch + 1) * CW]


def _final_body(s_ref, g_ref, dinv_ref, b_ref, o_ref):
    dinv = dinv_ref[:, :1]
    sc = jnp.concatenate([s_ref[i] for i in range(NCH)], axis=1)
    gc = jnp.concatenate([g_ref[i] for i in range(NCH)], axis=1)
    o_ref[...] = (sc + gc) * dinv + b_ref[...]


def _chunk_spec():
    return pl.BlockSpec((NCH, BLK, CW), lambda i: (0, i, 0))


def kernel(z, edge_index, W1, b1, W2, b2):
    f32 = jnp.float32
    z_p = jnp.zeros((NPAD, z.shape[1]), f32).at[:N].set(z)
    ei = jnp.pad(edge_index, ((0, 0), (0, EPAD - E)), constant_values=N)
    src = ei[0]
    dst = ei[1]
    dst16 = dst.reshape(16, NB, 128)
    src16 = src.reshape(16, NB, 128)

    ones1 = jnp.ones((128, 1), f32)
    zeros1 = jnp.zeros((ROWS_PER_TILE, 1), f32)
    zeros_cw = jnp.zeros((ROWS_PER_TILE, CW), f32)

    deg1 = _deg_kernel(dst16, ones1, zeros1)

    dinv8, g1 = pl.pallas_call(
        _mm_scale_body,
        grid=(GRID,),
        in_specs=[
            pl.BlockSpec((BLK, 64), lambda i: (i, 0)),
            pl.BlockSpec((64, 128), lambda i: (0, 0)),
            pl.BlockSpec((2, BLK, 1), lambda i: (0, i, 0)),
        ],
        out_specs=[
            pl.BlockSpec((BLK, 8), lambda i: (i, 0)),
            _chunk_spec(),
        ],
        out_shape=[
            jax.ShapeDtypeStruct((NPAD, 8), f32),
            jax.ShapeDtypeStruct((NCH, NPAD, CW), f32),
        ],
    )(z_p, W1, deg1)

    s1 = _seg_kernel(g1, src16, dst16, zeros_cw)

    g2 = pl.pallas_call(
        _layer2_body,
        grid=(GRID,),
        in_specs=[
            _chunk_spec(),
            _chunk_spec(),
            pl.BlockSpec((BLK, 8), lambda i: (i, 0)),
            pl.BlockSpec((1, 128), lambda i: (0, 0)),
            pl.BlockSpec((128, 128), lambda i: (0, 0)),
        ],
        out_specs=_chunk_spec(),
        out_shape=jax.ShapeDtypeStruct((NCH, NPAD, CW), f32),
    )(s1, g1, dinv8, b1.reshape(1, 128), W2)

    s2 = _seg_kernel(g2, src16, dst16, zeros_cw)

    x_hat = pl.pallas_call(
        _final_body,
        grid=(GRID,),
        in_specs=[
            _chunk_spec(),
            _chunk_spec(),
            pl.BlockSpec((BLK, 8), lambda i: (i, 0)),
            pl.BlockSpec((1, 128), lambda i: (0, 0)),
        ],
        out_specs=pl.BlockSpec((BLK, 128), lambda i: (i, 0)),
        out_shape=jax.ShapeDtypeStruct((NPAD, 128), f32),
    )(s2, g2, dinv8, b2.reshape(1, 128))

    return x_hat[:N]


# speedup vs baseline: 5.6520x; 1.0563x over previous
import functools

import jax
import jax.numpy as jnp
from jax import lax
from jax.experimental import pallas as pl
from jax.experimental.pallas import tpu as pltpu
from jax.experimental.pallas import tpu_sc as plsc

N = 50000
NPAD = 50176
E = 800000
EPAD = 802816
NB = 392
ROWS_PER_TILE = NPAD // 16
BLK = 512
GRID = NPAD // BLK
NCH = 16
CW = 128 // NCH
RING = 4
NGRP = NB // RING

_mesh = plsc.VectorSubcoreMesh(core_axis_name="c", subcore_axis_name="s")


@functools.partial(
    pl.kernel,
    out_type=jax.ShapeDtypeStruct((2, NPAD, 1), jnp.float32),
    mesh=_mesh,
    scratch_types=[
        pltpu.VMEM((NB, 128), jnp.int32),
        pltpu.VMEM((128, 1), jnp.float32),
        pltpu.VMEM_SHARED((NPAD, 1), jnp.float32),
    ],
    compiler_params=pltpu.CompilerParams(use_tc_tiling_on_sc=False),
)
def _deg_kernel(dst_hbm, ones_hbm, zeros_hbm, out_hbm, idx_v, ones_v, acc):
    c = lax.axis_index("c")
    s = lax.axis_index("s")
    pltpu.sync_copy(zeros_hbm, acc.at[pl.ds(s * ROWS_PER_TILE, ROWS_PER_TILE)])
    pltpu.sync_copy(ones_hbm, ones_v)
    pltpu.sync_copy(dst_hbm.at[s], idx_v)
    plsc.subcore_barrier()

    def body(j, carry):
        pltpu.sync_copy(ones_v, acc.at[idx_v.at[j]], add=True)
        return carry

    lax.fori_loop(0, NB, body, 0)
    plsc.subcore_barrier()
    sl = pl.ds(s * ROWS_PER_TILE, ROWS_PER_TILE)
    pltpu.sync_copy(acc.at[sl], out_hbm.at[c, sl])


@functools.partial(
    pl.kernel,
    out_type=jax.ShapeDtypeStruct((NCH, NPAD, CW), jnp.float32),
    mesh=_mesh,
    scratch_types=[
        pltpu.VMEM((NB, 128), jnp.int32),
        pltpu.VMEM((NB, 128), jnp.int32),
    ] + [pltpu.VMEM((128, CW), jnp.float32) for _ in range(RING)] + [
        pltpu.VMEM_SHARED((NPAD, CW), jnp.float32),
        pltpu.SemaphoreType.DMA,
        pltpu.SemaphoreType.DMA,
    ],
    compiler_params=pltpu.CompilerParams(use_tc_tiling_on_sc=False),
)
def _seg_kernel(gt_hbm, src_hbm, dst_hbm, zeros_hbm, out_hbm,
                src_v, dst_v, *rest):
    rows = rest[:RING]
    acc, gsem, ssem = rest[RING], rest[RING + 1], rest[RING + 2]
    c = lax.axis_index("c")
    s = lax.axis_index("s")
    sl = pl.ds(s * ROWS_PER_TILE, ROWS_PER_TILE)
    pltpu.sync_copy(src_hbm.at[s], src_v)
    pltpu.sync_copy(dst_hbm.at[s], dst_v)
    for k in range(NCH // 2):
        chunk = c * (NCH // 2) + k
        pltpu.sync_copy(zeros_hbm, acc.at[sl])
        plsc.subcore_barrier()

        def grp(g, carry):
            ds = [pltpu.async_copy(gt_hbm.at[chunk].at[src_v.at[g * RING + r]],
                                   rows[r], gsem) for r in range(RING)]
            ss = []
            for r in range(RING):
                ds[r].wait()
                ss.append(pltpu.async_copy(
                    rows[r], acc.at[dst_v.at[g * RING + r]], ssem, add=True))
            for d in ss:
                d.wait()
            return carry

        lax.fori_loop(0, NGRP, grp, 0)
        plsc.subcore_barrier()
        pltpu.sync_copy(acc.at[sl], out_hbm.at[chunk, sl])


def _mm_scale_body(z_ref, w_ref, deg_ref, dinv_ref, g_ref):
    d = deg_ref[...]
    dinv = lax.rsqrt(d[0, :, :1] + 1.0)
    dinv_ref[...] = jnp.broadcast_to(dinv, (BLK, 8))
    h = jnp.dot(z_ref[...], w_ref[...], preferred_element_type=jnp.float32)
    g = h * dinv
    for cch in range(NCH):
        g_ref[cch, :, :] = g[:, cch * CW:(cch + 1) * CW]


def _layer2_body(s_ref, g_ref, dinv_ref, b_ref, w_ref, g2_ref):
    dinv = dinv_ref[:, :1]
    sc = jnp.concatenate([s_ref[i] for i in range(NCH)], axis=1)
    gc = jnp.concatenate([g_ref[i] for i in range(NCH)], axis=1)
    x = jnp.maximum((sc + gc) * dinv + b_ref[...], 0.0)
    h2 = jnp.dot(x, w_ref[...], preferred_element_type=jnp.float32)
    g2 = h2 * dinv
    for cch in range(NCH):
        g2_ref[cch, :, :] = g2[:, cch * CW:(cch + 1) * CW]


def _final_body(s_ref, g_ref, dinv_ref, b_ref, o_ref):
    dinv = dinv_ref[:, :1]
    sc = jnp.concatenate([s_ref[i] for i in range(NCH)], axis=1)
    gc = jnp.concatenate([g_ref[i] for i in range(NCH)], axis=1)
    o_ref[...] = (sc + gc) * dinv + b_ref[...]


def _chunk_spec():
    return pl.BlockSpec((NCH, BLK, CW), lambda i: (0, i, 0))


def kernel(z, edge_index, W1, b1, W2, b2):
    f32 = jnp.float32
    z_p = jnp.zeros((NPAD, z.shape[1]), f32).at[:N].set(z)
    ei = jnp.pad(edge_index, ((0, 0), (0, EPAD - E)), constant_values=N)
    src = ei[0]
    dst = ei[1]
    dst16 = dst.reshape(16, NB, 128)
    src16 = src.reshape(16, NB, 128)

    ones1 = jnp.ones((128, 1), f32)
    zeros1 = jnp.zeros((ROWS_PER_TILE, 1), f32)
    zeros_cw = jnp.zeros((ROWS_PER_TILE, CW), f32)

    deg1 = _deg_kernel(dst16, ones1, zeros1)

    dinv8, g1 = pl.pallas_call(
        _mm_scale_body,
        grid=(GRID,),
        in_specs=[
            pl.BlockSpec((BLK, 64), lambda i: (i, 0)),
            pl.BlockSpec((64, 128), lambda i: (0, 0)),
            pl.BlockSpec((2, BLK, 1), lambda i: (0, i, 0)),
        ],
        out_specs=[
            pl.BlockSpec((BLK, 8), lambda i: (i, 0)),
            _chunk_spec(),
        ],
        out_shape=[
            jax.ShapeDtypeStruct((NPAD, 8), f32),
            jax.ShapeDtypeStruct((NCH, NPAD, CW), f32),
        ],
    )(z_p, W1, deg1)

    s1 = _seg_kernel(g1, src16, dst16, zeros_cw)

    g2 = pl.pallas_call(
        _layer2_body,
        grid=(GRID,),
        in_specs=[
            _chunk_spec(),
            _chunk_spec(),
            pl.BlockSpec((BLK, 8), lambda i: (i, 0)),
            pl.BlockSpec((1, 128), lambda i: (0, 0)),
            pl.BlockSpec((128, 128), lambda i: (0, 0)),
        ],
        out_specs=_chunk_spec(),
        out_shape=jax.ShapeDtypeStruct((NCH, NPAD, CW), f32),
    )(s1, g1, dinv8, b1.reshape(1, 128), W2)

    s2 = _seg_kernel(g2, src16, dst16, zeros_cw)

    x_hat = pl.pallas_call(
        _final_body,
        grid=(GRID,),
        in_specs=[
            _chunk_spec(),
            _chunk_spec(),
            pl.BlockSpec((BLK, 8), lambda i: (i, 0)),
            pl.BlockSpec((1, 128), lambda i: (0, 0)),
        ],
        out_specs=pl.BlockSpec((BLK, 128), lambda i: (i, 0)),
        out_shape=jax.ShapeDtypeStruct((NPAD, 128), f32),
    )(s2, g2, dinv8, b2.reshape(1, 128))

    return x_hat[:N]


# speedup vs baseline: 5.8572x; 1.0363x over previous
import functools

import jax
import jax.numpy as jnp
from jax import lax
from jax.experimental import pallas as pl
from jax.experimental.pallas import tpu as pltpu
from jax.experimental.pallas import tpu_sc as plsc

N = 50000
NPAD = 50176
E = 800000
EPAD = 802816
NB = 392
ROWS_PER_TILE = NPAD // 16
BLK = 512
GRID = NPAD // BLK
NCH = 16
CW = 128 // NCH
RING = 4
NGRP = NB // RING

_mesh = plsc.VectorSubcoreMesh(core_axis_name="c", subcore_axis_name="s")


@functools.partial(
    pl.kernel,
    out_type=jax.ShapeDtypeStruct((2, NPAD, 1), jnp.float32),
    mesh=_mesh,
    scratch_types=[
        pltpu.VMEM((NB, 128), jnp.int32),
        pltpu.VMEM((128, 1), jnp.float32),
        pltpu.VMEM_SHARED((NPAD, 1), jnp.float32),
    ],
    compiler_params=pltpu.CompilerParams(use_tc_tiling_on_sc=False),
)
def _deg_kernel(dst_hbm, ones_hbm, zeros_hbm, out_hbm, idx_v, ones_v, acc):
    c = lax.axis_index("c")
    s = lax.axis_index("s")
    pltpu.sync_copy(zeros_hbm, acc.at[pl.ds(s * ROWS_PER_TILE, ROWS_PER_TILE)])
    pltpu.sync_copy(ones_hbm, ones_v)
    pltpu.sync_copy(dst_hbm.at[s], idx_v)
    plsc.subcore_barrier()

    def body(j, carry):
        pltpu.sync_copy(ones_v, acc.at[idx_v.at[j]], add=True)
        return carry

    lax.fori_loop(0, NB, body, 0)
    plsc.subcore_barrier()
    sl = pl.ds(s * ROWS_PER_TILE, ROWS_PER_TILE)
    pltpu.sync_copy(acc.at[sl], out_hbm.at[c, sl])


@functools.partial(
    pl.kernel,
    out_type=jax.ShapeDtypeStruct((NCH, NPAD, CW), jnp.float32),
    mesh=_mesh,
    scratch_types=[
        pltpu.VMEM((NB, 128), jnp.int32),
        pltpu.VMEM((NB, 128), jnp.int32),
    ] + [pltpu.VMEM((128, CW), jnp.float32) for _ in range(RING)] + [
        pltpu.VMEM_SHARED((NPAD, CW), jnp.float32),
        pltpu.SemaphoreType.DMA,
        pltpu.SemaphoreType.DMA,
    ],
    compiler_params=pltpu.CompilerParams(use_tc_tiling_on_sc=False),
)
def _seg_kernel(gt_hbm, src_hbm, dst_hbm, zeros_hbm, out_hbm,
                src_v, dst_v, *rest):
    rows = rest[:RING]
    acc, gsem, ssem = rest[RING], rest[RING + 1], rest[RING + 2]
    c = lax.axis_index("c")
    s = lax.axis_index("s")
    sl = pl.ds(s * ROWS_PER_TILE, ROWS_PER_TILE)
    pltpu.sync_copy(src_hbm.at[s], src_v)
    pltpu.sync_copy(dst_hbm.at[s], dst_v)
    for k in range(NCH // 2):
        chunk = c * (NCH // 2) + k
        pltpu.sync_copy(zeros_hbm, acc.at[sl])
        plsc.subcore_barrier()

        ds = [pltpu.async_copy(gt_hbm.at[chunk].at[src_v.at[r]],
                               rows[r], gsem) for r in range(RING)]
        for r in range(RING):
            ds[r].wait()
            pltpu.async_copy(rows[r], acc.at[dst_v.at[r]], ssem, add=True)

        def grp(g, carry):
            ds = []
            for r in range(RING):
                pltpu.make_async_copy(rows[r], acc.at[dst_v.at[0]],
                                      ssem).wait()
                ds.append(pltpu.async_copy(
                    gt_hbm.at[chunk].at[src_v.at[g * RING + r]],
                    rows[r], gsem))
            for r in range(RING):
                ds[r].wait()
                pltpu.async_copy(rows[r], acc.at[dst_v.at[g * RING + r]],
                                 ssem, add=True)
            return carry

        lax.fori_loop(1, NGRP, grp, 0)
        for r in range(RING):
            pltpu.make_async_copy(rows[r], acc.at[dst_v.at[0]], ssem).wait()
        plsc.subcore_barrier()
        pltpu.sync_copy(acc.at[sl], out_hbm.at[chunk, sl])


def _mm_scale_body(z_ref, w_ref, deg_ref, dinv_ref, g_ref):
    d = deg_ref[...]
    dinv = lax.rsqrt(d[0, :, :1] + 1.0)
    dinv_ref[...] = jnp.broadcast_to(dinv, (BLK, 8))
    h = jnp.dot(z_ref[...], w_ref[...], preferred_element_type=jnp.float32)
    g = h * dinv
    for cch in range(NCH):
        g_ref[cch, :, :] = g[:, cch * CW:(cch + 1) * CW]


def _layer2_body(s_ref, g_ref, dinv_ref, b_ref, w_ref, g2_ref):
    dinv = dinv_ref[:, :1]
    sc = jnp.concatenate([s_ref[i] for i in range(NCH)], axis=1)
    gc = jnp.concatenate([g_ref[i] for i in range(NCH)], axis=1)
    x = jnp.maximum((sc + gc) * dinv + b_ref[...], 0.0)
    h2 = jnp.dot(x, w_ref[...], preferred_element_type=jnp.float32)
    g2 = h2 * dinv
    for cch in range(NCH):
        g2_ref[cch, :, :] = g2[:, cch * CW:(cch + 1) * CW]


def _final_body(s_ref, g_ref, dinv_ref, b_ref, o_ref):
    dinv = dinv_ref[:, :1]
    sc = jnp.concatenate([s_ref[i] for i in range(NCH)], axis=1)
    gc = jnp.concatenate([g_ref[i] for i in range(NCH)], axis=1)
    o_ref[...] = (sc + gc) * dinv + b_ref[...]


def _chunk_spec():
    return pl.BlockSpec((NCH, BLK, CW), lambda i: (0, i, 0))


def kernel(z, edge_index, W1, b1, W2, b2):
    f32 = jnp.float32
    z_p = jnp.zeros((NPAD, z.shape[1]), f32).at[:N].set(z)
    ei = jnp.pad(edge_index, ((0, 0), (0, EPAD - E)), constant_values=N)
    src = ei[0]
    dst = ei[1]
    dst16 = dst.reshape(16, NB, 128)
    src16 = src.reshape(16, NB, 128)

    ones1 = jnp.ones((128, 1), f32)
    zeros1 = jnp.zeros((ROWS_PER_TILE, 1), f32)
    zeros_cw = jnp.zeros((ROWS_PER_TILE, CW), f32)

    deg1 = _deg_kernel(dst16, ones1, zeros1)

    dinv8, g1 = pl.pallas_call(
        _mm_scale_body,
        grid=(GRID,),
        in_specs=[
            pl.BlockSpec((BLK, 64), lambda i: (i, 0)),
            pl.BlockSpec((64, 128), lambda i: (0, 0)),
            pl.BlockSpec((2, BLK, 1), lambda i: (0, i, 0)),
        ],
        out_specs=[
            pl.BlockSpec((BLK, 8), lambda i: (i, 0)),
            _chunk_spec(),
        ],
        out_shape=[
            jax.ShapeDtypeStruct((NPAD, 8), f32),
            jax.ShapeDtypeStruct((NCH, NPAD, CW), f32),
        ],
    )(z_p, W1, deg1)

    s1 = _seg_kernel(g1, src16, dst16, zeros_cw)

    g2 = pl.pallas_call(
        _layer2_body,
        grid=(GRID,),
        in_specs=[
            _chunk_spec(),
            _chunk_spec(),
            pl.BlockSpec((BLK, 8), lambda i: (i, 0)),
            pl.BlockSpec((1, 128), lambda i: (0, 0)),
            pl.BlockSpec((128, 128), lambda i: (0, 0)),
        ],
        out_specs=_chunk_spec(),
        out_shape=jax.ShapeDtypeStruct((NCH, NPAD, CW), f32),
    )(s1, g1, dinv8, b1.reshape(1, 128), W2)

    s2 = _seg_kernel(g2, src16, dst16, zeros_cw)

    x_hat = pl.pallas_call(
        _final_body,
        grid=(GRID,),
        in_specs=[
            _chunk_spec(),
            _chunk_spec(),
            pl.BlockSpec((BLK, 8), lambda i: (i, 0)),
            pl.BlockSpec((1, 128), lambda i: (0, 0)),
        ],
        out_specs=pl.BlockSpec((BLK, 128), lambda i: (i, 0)),
        out_shape=jax.ShapeDtypeStruct((NPAD, 128), f32),
    )(s2, g2, dinv8, b2.reshape(1, 128))

    return x_hat[:N]


# speedup vs baseline: 10.3871x; 1.7734x over previous
import functools

import jax
import jax.numpy as jnp
from jax import lax
from jax.experimental import pallas as pl
from jax.experimental.pallas import tpu as pltpu
from jax.experimental.pallas import tpu_sc as plsc

N = 50000
NPAD = 50176
E = 800000
EPAD = 802816
NB = 392
ROWS_PER_TILE = NPAD // 16
BLK = 512
GRID = NPAD // BLK
NCH = 8
CW = 128 // NCH
RING = 4
NBH = NB // 2
NGRP = NBH // RING

_mesh = plsc.VectorSubcoreMesh(core_axis_name="c", subcore_axis_name="s")


@functools.partial(
    pl.kernel,
    out_type=jax.ShapeDtypeStruct((2, NPAD, 1), jnp.float32),
    mesh=_mesh,
    scratch_types=[
        pltpu.VMEM((NB, 128), jnp.int32),
        pltpu.VMEM((128, 1), jnp.float32),
        pltpu.VMEM_SHARED((NPAD, 1), jnp.float32),
    ],
    compiler_params=pltpu.CompilerParams(use_tc_tiling_on_sc=False),
)
def _deg_kernel(dst_hbm, ones_hbm, zeros_hbm, out_hbm, idx_v, ones_v, acc):
    c = lax.axis_index("c")
    s = lax.axis_index("s")
    pltpu.sync_copy(zeros_hbm, acc.at[pl.ds(s * ROWS_PER_TILE, ROWS_PER_TILE)])
    pltpu.sync_copy(ones_hbm, ones_v)
    pltpu.sync_copy(dst_hbm.at[s], idx_v)
    plsc.subcore_barrier()

    def body(j, carry):
        pltpu.sync_copy(ones_v, acc.at[idx_v.at[j]], add=True)
        return carry

    lax.fori_loop(0, NB, body, 0)
    plsc.subcore_barrier()
    sl = pl.ds(s * ROWS_PER_TILE, ROWS_PER_TILE)
    pltpu.sync_copy(acc.at[sl], out_hbm.at[c, sl])


@functools.partial(
    pl.kernel,
    out_type=jax.ShapeDtypeStruct((NCH, NPAD, CW), jnp.float32),
    mesh=_mesh,
    scratch_types=[
        pltpu.VMEM((NBH, 128), jnp.int32),
        pltpu.VMEM((NBH, 128), jnp.int32),
    ] + [pltpu.VMEM((128, CW), jnp.float32) for _ in range(RING)] + [
        pltpu.VMEM_SHARED((NPAD, CW), jnp.float32),
        pltpu.SemaphoreType.DMA,
        pltpu.SemaphoreType.DMA,
    ],
    compiler_params=pltpu.CompilerParams(use_tc_tiling_on_sc=False),
)
def _seg_kernel(gt_hbm, src_hbm, dst_hbm, zeros_hbm, out_hbm,
                src_v, dst_v, *rest):
    rows = rest[:RING]
    acc, gsem, ssem = rest[RING], rest[RING + 1], rest[RING + 2]
    c = lax.axis_index("c")
    s = lax.axis_index("s")
    sl = pl.ds(s * ROWS_PER_TILE, ROWS_PER_TILE)
    for k in range(NCH // 2):
        chunk = c * (NCH // 2) + k
        pltpu.sync_copy(zeros_hbm, acc.at[sl])
        plsc.subcore_barrier()
        for h in range(2):
            pltpu.sync_copy(src_hbm.at[s, pl.ds(h * NBH, NBH)], src_v)
            pltpu.sync_copy(dst_hbm.at[s, pl.ds(h * NBH, NBH)], dst_v)

            ds = [pltpu.async_copy(gt_hbm.at[chunk].at[src_v.at[r]],
                                   rows[r], gsem) for r in range(RING)]
            for r in range(RING):
                ds[r].wait()
                pltpu.async_copy(rows[r], acc.at[dst_v.at[r]], ssem,
                                 add=True)

            def grp(g, carry):
                ds = []
                for r in range(RING):
                    pltpu.make_async_copy(rows[r], acc.at[dst_v.at[0]],
                                          ssem).wait()
                    ds.append(pltpu.async_copy(
                        gt_hbm.at[chunk].at[src_v.at[g * RING + r]],
                        rows[r], gsem))
                for r in range(RING):
                    ds[r].wait()
                    pltpu.async_copy(rows[r],
                                     acc.at[dst_v.at[g * RING + r]],
                                     ssem, add=True)
                return carry

            lax.fori_loop(1, NGRP, grp, 0)
            for r in range(RING):
                pltpu.make_async_copy(rows[r], acc.at[dst_v.at[0]],
                                      ssem).wait()
        plsc.subcore_barrier()
        pltpu.sync_copy(acc.at[sl], out_hbm.at[chunk, sl])


def _mm_scale_body(z_ref, w_ref, deg_ref, dinv_ref, g_ref):
    d = deg_ref[...]
    dinv = lax.rsqrt(d[0, :, :1] + 1.0)
    dinv_ref[...] = jnp.broadcast_to(dinv, (BLK, 8))
    h = jnp.dot(z_ref[...], w_ref[...], preferred_element_type=jnp.float32)
    g = h * dinv
    for cch in range(NCH):
        g_ref[cch, :, :] = g[:, cch * CW:(cch + 1) * CW]


def _layer2_body(s_ref, g_ref, dinv_ref, b_ref, w_ref, g2_ref):
    dinv = dinv_ref[:, :1]
    sc = jnp.concatenate([s_ref[i] for i in range(NCH)], axis=1)
    gc = jnp.concatenate([g_ref[i] for i in range(NCH)], axis=1)
    x = jnp.maximum((sc + gc) * dinv + b_ref[...], 0.0)
    h2 = jnp.dot(x, w_ref[...], preferred_element_type=jnp.float32)
    g2 = h2 * dinv
    for cch in range(NCH):
        g2_ref[cch, :, :] = g2[:, cch * CW:(cch + 1) * CW]


def _final_body(s_ref, g_ref, dinv_ref, b_ref, o_ref):
    dinv = dinv_ref[:, :1]
    sc = jnp.concatenate([s_ref[i] for i in range(NCH)], axis=1)
    gc = jnp.concatenate([g_ref[i] for i in range(NCH)], axis=1)
    o_ref[...] = (sc + gc) * dinv + b_ref[...]


def _chunk_spec():
    return pl.BlockSpec((NCH, BLK, CW), lambda i: (0, i, 0))


def kernel(z, edge_index, W1, b1, W2, b2):
    f32 = jnp.float32
    z_p = jnp.zeros((NPAD, z.shape[1]), f32).at[:N].set(z)
    ei = jnp.pad(edge_index, ((0, 0), (0, EPAD - E)), constant_values=N)
    src = ei[0]
    dst = ei[1]
    dst16 = dst.reshape(16, NB, 128)
    src16 = src.reshape(16, NB, 128)

    ones1 = jnp.ones((128, 1), f32)
    zeros1 = jnp.zeros((ROWS_PER_TILE, 1), f32)
    zeros_cw = jnp.zeros((ROWS_PER_TILE, CW), f32)

    deg1 = _deg_kernel(dst16, ones1, zeros1)

    dinv8, g1 = pl.pallas_call(
        _mm_scale_body,
        grid=(GRID,),
        in_specs=[
            pl.BlockSpec((BLK, 64), lambda i: (i, 0)),
            pl.BlockSpec((64, 128), lambda i: (0, 0)),
            pl.BlockSpec((2, BLK, 1), lambda i: (0, i, 0)),
        ],
        out_specs=[
            pl.BlockSpec((BLK, 8), lambda i: (i, 0)),
            _chunk_spec(),
        ],
        out_shape=[
            jax.ShapeDtypeStruct((NPAD, 8), f32),
            jax.ShapeDtypeStruct((NCH, NPAD, CW), f32),
        ],
    )(z_p, W1, deg1)

    s1 = _seg_kernel(g1, src16, dst16, zeros_cw)

    g2 = pl.pallas_call(
        _layer2_body,
        grid=(GRID,),
        in_specs=[
            _chunk_spec(),
            _chunk_spec(),
            pl.BlockSpec((BLK, 8), lambda i: (i, 0)),
            pl.BlockSpec((1, 128), lambda i: (0, 0)),
            pl.BlockSpec((128, 128), lambda i: (0, 0)),
        ],
        out_specs=_chunk_spec(),
        out_shape=jax.ShapeDtypeStruct((NCH, NPAD, CW), f32),
    )(s1, g1, dinv8, b1.reshape(1, 128), W2)

    s2 = _seg_kernel(g2, src16, dst16, zeros_cw)

    x_hat = pl.pallas_call(
        _final_body,
        grid=(GRID,),
        in_specs=[
            _chunk_spec(),
            _chunk_spec(),
            pl.BlockSpec((BLK, 8), lambda i: (i, 0)),
            pl.BlockSpec((1, 128), lambda i: (0, 0)),
        ],
        out_specs=pl.BlockSpec((BLK, 128), lambda i: (i, 0)),
        out_shape=jax.ShapeDtypeStruct((NPAD, 128), f32),
    )(s2, g2, dinv8, b2.reshape(1, 128))

    return x_hat[:N]


# speedup vs baseline: 13.9515x; 1.3431x over previous
import functools

import jax
import jax.numpy as jnp
from jax import lax
from jax.experimental import pallas as pl
from jax.experimental.pallas import tpu as pltpu
from jax.experimental.pallas import tpu_sc as plsc

N = 50000
NPAD = 50176
E = 800000
EPAD = 802816
NB = 392
ROWS_PER_TILE = NPAD // 16
BLK = 512
GRID = NPAD // BLK
NCH = 4
CW = 128 // NCH
RING = 2
NBH = NB // 8
NGRP = NBH // RING

_mesh = plsc.VectorSubcoreMesh(core_axis_name="c", subcore_axis_name="s")


@functools.partial(
    pl.kernel,
    out_type=jax.ShapeDtypeStruct((2, NPAD, 1), jnp.float32),
    mesh=_mesh,
    scratch_types=[
        pltpu.VMEM((NB, 128), jnp.int32),
        pltpu.VMEM((128, 1), jnp.float32),
        pltpu.VMEM_SHARED((NPAD, 1), jnp.float32),
    ],
    compiler_params=pltpu.CompilerParams(use_tc_tiling_on_sc=False),
)
def _deg_kernel(dst_hbm, ones_hbm, zeros_hbm, out_hbm, idx_v, ones_v, acc):
    c = lax.axis_index("c")
    s = lax.axis_index("s")
    pltpu.sync_copy(zeros_hbm, acc.at[pl.ds(s * ROWS_PER_TILE, ROWS_PER_TILE)])
    pltpu.sync_copy(ones_hbm, ones_v)
    pltpu.sync_copy(dst_hbm.at[s], idx_v)
    plsc.subcore_barrier()

    def body(j, carry):
        pltpu.sync_copy(ones_v, acc.at[idx_v.at[j]], add=True)
        return carry

    lax.fori_loop(0, NB, body, 0)
    plsc.subcore_barrier()
    sl = pl.ds(s * ROWS_PER_TILE, ROWS_PER_TILE)
    pltpu.sync_copy(acc.at[sl], out_hbm.at[c, sl])


@functools.partial(
    pl.kernel,
    out_type=jax.ShapeDtypeStruct((NCH, NPAD, CW), jnp.float32),
    mesh=_mesh,
    scratch_types=[
        pltpu.VMEM((NBH, 128), jnp.int32),
        pltpu.VMEM((NBH, 128), jnp.int32),
    ] + [pltpu.VMEM((128, CW), jnp.float32) for _ in range(RING)] + [
        pltpu.VMEM_SHARED((NPAD, CW), jnp.float32),
        pltpu.SemaphoreType.DMA,
        pltpu.SemaphoreType.DMA,
    ],
    compiler_params=pltpu.CompilerParams(use_tc_tiling_on_sc=False),
)
def _seg_kernel(gt_hbm, src_hbm, dst_hbm, zeros_hbm, out_hbm,
                src_v, dst_v, *rest):
    rows = rest[:RING]
    acc, gsem, ssem = rest[RING], rest[RING + 1], rest[RING + 2]
    c = lax.axis_index("c")
    s = lax.axis_index("s")
    sl = pl.ds(s * ROWS_PER_TILE, ROWS_PER_TILE)
    for k in range(NCH // 2):
        chunk = c * (NCH // 2) + k
        pltpu.sync_copy(zeros_hbm, acc.at[sl])
        plsc.subcore_barrier()
        for h in range(8):
            pltpu.sync_copy(src_hbm.at[s, pl.ds(h * NBH, NBH)], src_v)
            pltpu.sync_copy(dst_hbm.at[s, pl.ds(h * NBH, NBH)], dst_v)

            ds = [pltpu.async_copy(gt_hbm.at[chunk].at[src_v.at[r]],
                                   rows[r], gsem) for r in range(RING)]
            for r in range(RING):
                ds[r].wait()
                pltpu.async_copy(rows[r], acc.at[dst_v.at[r]], ssem,
                                 add=True)

            def grp(g, carry):
                ds = []
                for r in range(RING):
                    pltpu.make_async_copy(rows[r], acc.at[dst_v.at[0]],
                                          ssem).wait()
                    ds.append(pltpu.async_copy(
                        gt_hbm.at[chunk].at[src_v.at[g * RING + r]],
                        rows[r], gsem))
                for r in range(RING):
                    ds[r].wait()
                    pltpu.async_copy(rows[r],
                                     acc.at[dst_v.at[g * RING + r]],
                                     ssem, add=True)
                return carry

            lax.fori_loop(1, NGRP, grp, 0)
            for r in range(RING):
                pltpu.make_async_copy(rows[r], acc.at[dst_v.at[0]],
                                      ssem).wait()
        plsc.subcore_barrier()
        pltpu.sync_copy(acc.at[sl], out_hbm.at[chunk, sl])


def _mm_scale_body(z_ref, w_ref, deg_ref, dinv_ref, g_ref):
    d = deg_ref[...]
    dinv = lax.rsqrt(d[0, :, :1] + 1.0)
    dinv_ref[...] = jnp.broadcast_to(dinv, (BLK, 8))
    h = jnp.dot(z_ref[...], w_ref[...], preferred_element_type=jnp.float32)
    g = h * dinv
    for cch in range(NCH):
        g_ref[cch, :, :] = g[:, cch * CW:(cch + 1) * CW]


def _layer2_body(s_ref, g_ref, dinv_ref, b_ref, w_ref, g2_ref):
    dinv = dinv_ref[:, :1]
    sc = jnp.concatenate([s_ref[i] for i in range(NCH)], axis=1)
    gc = jnp.concatenate([g_ref[i] for i in range(NCH)], axis=1)
    x = jnp.maximum((sc + gc) * dinv + b_ref[...], 0.0)
    h2 = jnp.dot(x, w_ref[...], preferred_element_type=jnp.float32)
    g2 = h2 * dinv
    for cch in range(NCH):
        g2_ref[cch, :, :] = g2[:, cch * CW:(cch + 1) * CW]


def _final_body(s_ref, g_ref, dinv_ref, b_ref, o_ref):
    dinv = dinv_ref[:, :1]
    sc = jnp.concatenate([s_ref[i] for i in range(NCH)], axis=1)
    gc = jnp.concatenate([g_ref[i] for i in range(NCH)], axis=1)
    o_ref[...] = (sc + gc) * dinv + b_ref[...]


def _chunk_spec():
    return pl.BlockSpec((NCH, BLK, CW), lambda i: (0, i, 0))


def kernel(z, edge_index, W1, b1, W2, b2):
    f32 = jnp.float32
    z_p = jnp.zeros((NPAD, z.shape[1]), f32).at[:N].set(z)
    ei = jnp.pad(edge_index, ((0, 0), (0, EPAD - E)), constant_values=N)
    src = ei[0]
    dst = ei[1]
    dst16 = dst.reshape(16, NB, 128)
    src16 = src.reshape(16, NB, 128)

    ones1 = jnp.ones((128, 1), f32)
    zeros1 = jnp.zeros((ROWS_PER_TILE, 1), f32)
    zeros_cw = jnp.zeros((ROWS_PER_TILE, CW), f32)

    deg1 = _deg_kernel(dst16, ones1, zeros1)

    dinv8, g1 = pl.pallas_call(
        _mm_scale_body,
        grid=(GRID,),
        in_specs=[
            pl.BlockSpec((BLK, 64), lambda i: (i, 0)),
            pl.BlockSpec((64, 128), lambda i: (0, 0)),
            pl.BlockSpec((2, BLK, 1), lambda i: (0, i, 0)),
        ],
        out_specs=[
            pl.BlockSpec((BLK, 8), lambda i: (i, 0)),
            _chunk_spec(),
        ],
        out_shape=[
            jax.ShapeDtypeStruct((NPAD, 8), f32),
            jax.ShapeDtypeStruct((NCH, NPAD, CW), f32),
        ],
    )(z_p, W1, deg1)

    s1 = _seg_kernel(g1, src16, dst16, zeros_cw)

    g2 = pl.pallas_call(
        _layer2_body,
        grid=(GRID,),
        in_specs=[
            _chunk_spec(),
            _chunk_spec(),
            pl.BlockSpec((BLK, 8), lambda i: (i, 0)),
            pl.BlockSpec((1, 128), lambda i: (0, 0)),
            pl.BlockSpec((128, 128), lambda i: (0, 0)),
        ],
        out_specs=_chunk_spec(),
        out_shape=jax.ShapeDtypeStruct((NCH, NPAD, CW), f32),
    )(s1, g1, dinv8, b1.reshape(1, 128), W2)

    s2 = _seg_kernel(g2, src16, dst16, zeros_cw)

    x_hat = pl.pallas_call(
        _final_body,
        grid=(GRID,),
        in_specs=[
            _chunk_spec(),
            _chunk_spec(),
            pl.BlockSpec((BLK, 8), lambda i: (i, 0)),
            pl.BlockSpec((1, 128), lambda i: (0, 0)),
        ],
        out_specs=pl.BlockSpec((BLK, 128), lambda i: (i, 0)),
        out_shape=jax.ShapeDtypeStruct((NPAD, 128), f32),
    )(s2, g2, dinv8, b2.reshape(1, 128))

    return x_hat[:N]


# speedup vs baseline: 14.7390x; 1.0564x over previous
import functools

import jax
import jax.numpy as jnp
from jax import lax
from jax.experimental import pallas as pl
from jax.experimental.pallas import tpu as pltpu
from jax.experimental.pallas import tpu_sc as plsc

N = 50000
NPAD = 50176
E = 800000
EPAD = 802816
NB = 392
ROWS_PER_TILE = NPAD // 16
BLK = 1024
GRID = NPAD // BLK
NCH = 4
CW = 128 // NCH
RING = 2
NBH = NB // 8
NGRP = NBH // RING

_mesh = plsc.VectorSubcoreMesh(core_axis_name="c", subcore_axis_name="s")


@functools.partial(
    pl.kernel,
    out_type=jax.ShapeDtypeStruct((2, NPAD, 1), jnp.float32),
    mesh=_mesh,
    scratch_types=[
        pltpu.VMEM((NB, 128), jnp.int32),
        pltpu.VMEM((128, 1), jnp.float32),
        pltpu.VMEM_SHARED((NPAD, 1), jnp.float32),
    ],
    compiler_params=pltpu.CompilerParams(use_tc_tiling_on_sc=False),
)
def _deg_kernel(dst_hbm, ones_hbm, zeros_hbm, out_hbm, idx_v, ones_v, acc):
    c = lax.axis_index("c")
    s = lax.axis_index("s")
    pltpu.sync_copy(zeros_hbm, acc.at[pl.ds(s * ROWS_PER_TILE, ROWS_PER_TILE)])
    pltpu.sync_copy(ones_hbm, ones_v)
    pltpu.sync_copy(dst_hbm.at[s], idx_v)
    plsc.subcore_barrier()

    def body(j, carry):
        pltpu.sync_copy(ones_v, acc.at[idx_v.at[j]], add=True)
        return carry

    lax.fori_loop(0, NB, body, 0)
    plsc.subcore_barrier()
    sl = pl.ds(s * ROWS_PER_TILE, ROWS_PER_TILE)
    pltpu.sync_copy(acc.at[sl], out_hbm.at[c, sl])


@functools.partial(
    pl.kernel,
    out_type=jax.ShapeDtypeStruct((NCH, NPAD, CW), jnp.float32),
    mesh=_mesh,
    scratch_types=[
        pltpu.VMEM((NBH, 128), jnp.int32),
        pltpu.VMEM((NBH, 128), jnp.int32),
    ] + [pltpu.VMEM((128, CW), jnp.float32) for _ in range(RING)] + [
        pltpu.VMEM_SHARED((NPAD, CW), jnp.float32),
        pltpu.SemaphoreType.DMA,
        pltpu.SemaphoreType.DMA,
    ],
    compiler_params=pltpu.CompilerParams(use_tc_tiling_on_sc=False),
)
def _seg_kernel(gt_hbm, src_hbm, dst_hbm, zeros_hbm, out_hbm,
                src_v, dst_v, *rest):
    rows = rest[:RING]
    acc, gsem, ssem = rest[RING], rest[RING + 1], rest[RING + 2]
    c = lax.axis_index("c")
    s = lax.axis_index("s")
    sl = pl.ds(s * ROWS_PER_TILE, ROWS_PER_TILE)
    for k in range(NCH // 2):
        chunk = c * (NCH // 2) + k
        pltpu.sync_copy(zeros_hbm, acc.at[sl])
        plsc.subcore_barrier()
        for h in range(8):
            pltpu.sync_copy(src_hbm.at[s, pl.ds(h * NBH, NBH)], src_v)
            pltpu.sync_copy(dst_hbm.at[s, pl.ds(h * NBH, NBH)], dst_v)

            ds = [pltpu.async_copy(gt_hbm.at[chunk].at[src_v.at[r]],
                                   rows[r], gsem) for r in range(RING)]
            for r in range(RING):
                ds[r].wait()
                pltpu.async_copy(rows[r], acc.at[dst_v.at[r]], ssem,
                                 add=True)

            def grp(g, carry):
                ds = []
                for r in range(RING):
                    pltpu.make_async_copy(rows[r], acc.at[dst_v.at[0]],
                                          ssem).wait()
                    ds.append(pltpu.async_copy(
                        gt_hbm.at[chunk].at[src_v.at[g * RING + r]],
                        rows[r], gsem))
                for r in range(RING):
                    ds[r].wait()
                    pltpu.async_copy(rows[r],
                                     acc.at[dst_v.at[g * RING + r]],
                                     ssem, add=True)
                return carry

            lax.fori_loop(1, NGRP, grp, 0)
            for r in range(RING):
                pltpu.make_async_copy(rows[r], acc.at[dst_v.at[0]],
                                      ssem).wait()
        plsc.subcore_barrier()
        pltpu.sync_copy(acc.at[sl], out_hbm.at[chunk, sl])


def _mm_scale_body(z_ref, w_ref, deg_ref, dinv_ref, g_ref):
    d = deg_ref[...]
    dinv = lax.rsqrt(d[0, :, :1] + 1.0)
    dinv_ref[...] = jnp.broadcast_to(dinv, (BLK, 8))
    h = jnp.dot(z_ref[...], w_ref[...], preferred_element_type=jnp.float32)
    g = h * dinv
    for cch in range(NCH):
        g_ref[cch, :, :] = g[:, cch * CW:(cch + 1) * CW]


def _layer2_body(s_ref, g_ref, dinv_ref, b_ref, w_ref, g2_ref):
    dinv = dinv_ref[:, :1]
    sc = jnp.concatenate([s_ref[i] for i in range(NCH)], axis=1)
    gc = jnp.concatenate([g_ref[i] for i in range(NCH)], axis=1)
    x = jnp.maximum((sc + gc) * dinv + b_ref[...], 0.0)
    h2 = jnp.dot(x, w_ref[...], preferred_element_type=jnp.float32)
    g2 = h2 * dinv
    for cch in range(NCH):
        g2_ref[cch, :, :] = g2[:, cch * CW:(cch + 1) * CW]


def _final_body(s_ref, g_ref, dinv_ref, b_ref, o_ref):
    dinv = dinv_ref[:, :1]
    sc = jnp.concatenate([s_ref[i] for i in range(NCH)], axis=1)
    gc = jnp.concatenate([g_ref[i] for i in range(NCH)], axis=1)
    o_ref[...] = (sc + gc) * dinv + b_ref[...]


def _chunk_spec():
    return pl.BlockSpec((NCH, BLK, CW), lambda i: (0, i, 0))


def kernel(z, edge_index, W1, b1, W2, b2):
    f32 = jnp.float32
    z_p = jnp.zeros((NPAD, z.shape[1]), f32).at[:N].set(z)
    ei = jnp.pad(edge_index, ((0, 0), (0, EPAD - E)), constant_values=N)
    src = ei[0]
    dst = ei[1]
    dst16 = dst.reshape(16, NB, 128)
    src16 = src.reshape(16, NB, 128)

    ones1 = jnp.ones((128, 1), f32)
    zeros1 = jnp.zeros((ROWS_PER_TILE, 1), f32)
    zeros_cw = jnp.zeros((ROWS_PER_TILE, CW), f32)

    deg1 = _deg_kernel(dst16, ones1, zeros1)

    dinv8, g1 = pl.pallas_call(
        _mm_scale_body,
        grid=(GRID,),
        in_specs=[
            pl.BlockSpec((BLK, 64), lambda i: (i, 0)),
            pl.BlockSpec((64, 128), lambda i: (0, 0)),
            pl.BlockSpec((2, BLK, 1), lambda i: (0, i, 0)),
        ],
        out_specs=[
            pl.BlockSpec((BLK, 8), lambda i: (i, 0)),
            _chunk_spec(),
        ],
        out_shape=[
            jax.ShapeDtypeStruct((NPAD, 8), f32),
            jax.ShapeDtypeStruct((NCH, NPAD, CW), f32),
        ],
    )(z_p, W1, deg1)

    s1 = _seg_kernel(g1, src16, dst16, zeros_cw)

    g2 = pl.pallas_call(
        _layer2_body,
        grid=(GRID,),
        in_specs=[
            _chunk_spec(),
            _chunk_spec(),
            pl.BlockSpec((BLK, 8), lambda i: (i, 0)),
            pl.BlockSpec((1, 128), lambda i: (0, 0)),
            pl.BlockSpec((128, 128), lambda i: (0, 0)),
        ],
        out_specs=_chunk_spec(),
        out_shape=jax.ShapeDtypeStruct((NCH, NPAD, CW), f32),
    )(s1, g1, dinv8, b1.reshape(1, 128), W2)

    s2 = _seg_kernel(g2, src16, dst16, zeros_cw)

    x_hat = pl.pallas_call(
        _final_body,
        grid=(GRID,),
        in_specs=[
            _chunk_spec(),
            _chunk_spec(),
            pl.BlockSpec((BLK, 8), lambda i: (i, 0)),
            pl.BlockSpec((1, 128), lambda i: (0, 0)),
        ],
        out_specs=pl.BlockSpec((BLK, 128), lambda i: (i, 0)),
        out_shape=jax.ShapeDtypeStruct((NPAD, 128), f32),
    )(s2, g2, dinv8, b2.reshape(1, 128))

    return x_hat[:N]
